# baseline (speedup 1.0000x reference)
import math

import jax
import jax.numpy as jnp
from jax.experimental import pallas as pl

_B, _K, _D = 4096, 133, 2
_TOPK = 8
_LOG_RAMP = math.log(math.sqrt(2.0 * math.pi))
_SQRT2 = math.sqrt(2.0)
_ORI_WEIGHT = 1.0
_OHKM_WEIGHT = 0.5
_GRID = 2
_SB = 64 // _GRID // 2
_NGROUPS = 17

_SORT8 = (
    (0, 1), (2, 3), (4, 5), (6, 7),
    (0, 2), (1, 3), (4, 6), (5, 7),
    (1, 2), (5, 6), (0, 4), (3, 7),
    (1, 5), (2, 6),
    (1, 4), (3, 6),
    (2, 4), (3, 5),
    (3, 4),
)
_BITONIC8 = (
    (0, 4), (1, 5), (2, 6), (3, 7),
    (0, 2), (1, 3), (4, 6), (5, 7),
    (0, 1), (2, 3), (4, 5), (6, 7),
)


def _ce(lst, i, j):
    a, b = lst[i], lst[j]
    lst[i] = jnp.maximum(a, b)
    lst[j] = jnp.minimum(a, b)


def _merge_top8(a, b):
    l = [jnp.maximum(a[i], b[7 - i]) for i in range(8)]
    for i, j in _BITONIC8:
        _ce(l, i, j)
    return l


def _process(pred_ref, sigma_ref, nf_ref, tgt_ref, w_ref):
    neg = jnp.float32(-jnp.inf)
    pad = jnp.full((_SB, 128), neg, jnp.float32)

    ol_acc = jnp.zeros((_SB, 128), jnp.float32)
    top = None
    for g in range(_NGROUPS):
        j0 = 8 * g
        nj = min(8, _K - j0)
        sl = pl.ds(j0, nj)
        pred = pred_ref[sl]
        sigma = sigma_ref[sl]
        nf = nf_ref[sl]
        tgt = tgt_ref[sl]
        w = w_ref[sl]

        q = jnp.log(sigma) + _LOG_RAMP + jnp.abs(tgt - pred) / (_SQRT2 * sigma + 1e-9)
        ori = nf + q
        ow = ori * w
        for j in range(nj):
            ol_acc = ol_acc + ow[j]

        v = jnp.where(w == 0.0, neg, ori)
        grp = [v[j] for j in range(nj)] + [pad] * (8 - nj)
        for i, j in _SORT8:
            _ce(grp, i, j)
        top = grp if top is None else _merge_top8(top, grp)

    tsum = jnp.zeros((_SB, 128), jnp.float32)
    for i in range(_TOPK):
        tsum = tsum + jnp.where(top[i] > neg, top[i], 0.0)
    return _ORI_WEIGHT * jnp.sum(ol_acc) + _OHKM_WEIGHT * jnp.sum(tsum)


def _loss_kernel(pa, sa, na, ta, wa, pb, sb, nb, tb, wb, out_ref):
    total = _process(pa, sa, na, ta, wa) + _process(pb, sb, nb, tb, wb)
    total2d = total * jnp.ones((1, 1), jnp.float32)

    @pl.when(pl.program_id(0) == 0)
    def _init():
        out_ref[...] = jnp.zeros_like(out_ref)

    out_ref[...] += total2d


def _native_view(x):
    return x.reshape(32, 128, _K, _D).transpose(2, 0, 3, 1).reshape(_K, 64, 128)


def kernel(pred_jts, sigma, nf_loss, target_uv, target_uv_weight):
    views = [_native_view(a) for a in
             (pred_jts, sigma, nf_loss, target_uv, target_uv_weight)]
    args = views + views
    spec_a = pl.BlockSpec((_K, _SB, 128), lambda i: (0, i, 0))
    spec_b = pl.BlockSpec((_K, _SB, 128), lambda i: (0, i + _GRID, 0))
    out = pl.pallas_call(
        _loss_kernel,
        grid=(_GRID,),
        in_specs=[spec_a] * 5 + [spec_b] * 5,
        out_specs=pl.BlockSpec((1, 1), lambda i: (0, 0)),
        out_shape=jax.ShapeDtypeStruct((1, 1), jnp.float32),
    )(*args)
    return (out[0, 0] / _B).astype(jnp.float32)


# speedup vs baseline: 1.0209x; 1.0209x over previous
import math

import jax
import jax.numpy as jnp
from jax.experimental import pallas as pl

_B, _K, _D = 4096, 133, 2
_TOPK = 8
_LOG_RAMP = math.log(math.sqrt(2.0 * math.pi))
_SQRT2 = math.sqrt(2.0)
_ORI_WEIGHT = 1.0
_OHKM_WEIGHT = 0.5
_GRID = 4
_SB = 64 // _GRID // 2
_NGROUPS = 17

_SORT8 = (
    (0, 1), (2, 3), (4, 5), (6, 7),
    (0, 2), (1, 3), (4, 6), (5, 7),
    (1, 2), (5, 6), (0, 4), (3, 7),
    (1, 5), (2, 6),
    (1, 4), (3, 6),
    (2, 4), (3, 5),
    (3, 4),
)
_BITONIC8 = (
    (0, 4), (1, 5), (2, 6), (3, 7),
    (0, 2), (1, 3), (4, 6), (5, 7),
    (0, 1), (2, 3), (4, 5), (6, 7),
)


def _ce(lst, i, j):
    a, b = lst[i], lst[j]
    lst[i] = jnp.maximum(a, b)
    lst[j] = jnp.minimum(a, b)


def _merge_top8(a, b):
    l = [jnp.maximum(a[i], b[7 - i]) for i in range(8)]
    for i, j in _BITONIC8:
        _ce(l, i, j)
    return l


def _process(pred_ref, sigma_ref, nf_ref, tgt_ref, w_ref):
    neg = jnp.float32(-jnp.inf)
    pad = jnp.full((_SB, 128), neg, jnp.float32)

    ol_acc = jnp.zeros((_SB, 128), jnp.float32)
    top = None
    for g in range(_NGROUPS):
        j0 = 8 * g
        nj = min(8, _K - j0)
        sl = pl.ds(j0, nj)
        pred = pred_ref[sl]
        sigma = sigma_ref[sl]
        nf = nf_ref[sl]
        tgt = tgt_ref[sl]
        w = w_ref[sl]

        q = jnp.log(sigma) + _LOG_RAMP + jnp.abs(tgt - pred) / (_SQRT2 * sigma + 1e-9)
        ori = nf + q
        ow = ori * w
        for j in range(nj):
            ol_acc = ol_acc + ow[j]

        v = jnp.where(w == 0.0, neg, ori)
        grp = [v[j] for j in range(nj)] + [pad] * (8 - nj)
        for i, j in _SORT8:
            _ce(grp, i, j)
        top = grp if top is None else _merge_top8(top, grp)

    tsum = jnp.zeros((_SB, 128), jnp.float32)
    for i in range(_TOPK):
        tsum = tsum + jnp.where(top[i] > neg, top[i], 0.0)
    return _ORI_WEIGHT * jnp.sum(ol_acc) + _OHKM_WEIGHT * jnp.sum(tsum)


def _loss_kernel(pa, sa, na, ta, wa, pb, sb, nb, tb, wb, out_ref):
    total = _process(pa, sa, na, ta, wa) + _process(pb, sb, nb, tb, wb)
    total2d = total * jnp.ones((1, 1), jnp.float32)

    @pl.when(pl.program_id(0) == 0)
    def _init():
        out_ref[...] = jnp.zeros_like(out_ref)

    out_ref[...] += total2d


def _native_view(x):
    return x.reshape(32, 128, _K, _D).transpose(2, 0, 3, 1).reshape(_K, 64, 128)


def kernel(pred_jts, sigma, nf_loss, target_uv, target_uv_weight):
    views = [_native_view(a) for a in
             (pred_jts, sigma, nf_loss, target_uv, target_uv_weight)]
    args = views + views
    spec_a = pl.BlockSpec((_K, _SB, 128), lambda i: (0, i, 0))
    spec_b = pl.BlockSpec((_K, _SB, 128), lambda i: (0, i + _GRID, 0))
    out = pl.pallas_call(
        _loss_kernel,
        grid=(_GRID,),
        in_specs=[spec_a] * 5 + [spec_b] * 5,
        out_specs=pl.BlockSpec((1, 1), lambda i: (0, 0)),
        out_shape=jax.ShapeDtypeStruct((1, 1), jnp.float32),
    )(*args)
    return (out[0, 0] / _B).astype(jnp.float32)
